# Initial kernel scaffold; baseline (speedup 1.0000x reference)
import jax
import jax.numpy as jnp
from jax.experimental import pallas as pl


def kernel(power_spectrum, all_species, segment_ids, weights):
    raise NotImplementedError("write your pallas kernel here")


# speedup vs baseline: 2.2095x; 2.2095x over previous
import jax
import jax.numpy as jnp
from jax import lax
from jax.experimental import pallas as pl

N = 320000
D = 128
NUM_SEGMENTS = 4096
BLK = 512
W = 512
NUM_WIN = NUM_SEGMENTS // W


def _body(ids_ref, p_ref, w_ref, out_ref):
    @pl.when(pl.program_id(0) == 0)
    def _():
        out_ref[...] = jnp.zeros_like(out_ref)

    p = p_ref[...]
    wts = w_ref[...]
    y = lax.dot_general(p, wts, (((1,), (1,)), ((), ())),
                        preferred_element_type=jnp.float32)

    ids = ids_ref[0]
    first = jnp.min(ids)
    last = jnp.max(ids)
    w_lo = first // W
    w_hi = last // W

    def win_body(k, _):
        base = k * W
        seg = base + lax.broadcasted_iota(jnp.int32, (W, BLK), 0)
        onehot = (seg == ids).astype(jnp.float32)
        contrib = lax.dot_general(y, onehot, (((0,), (1,)), ((), ())),
                                  preferred_element_type=jnp.float32)
        out_ref[pl.ds(k, 1), :] += contrib
        return 0

    lax.fori_loop(w_lo, w_hi + 1, win_body, 0)


def kernel(power_spectrum, all_species, segment_ids, weights):
    del all_species
    ids = segment_ids.astype(jnp.int32).reshape(N // BLK, 1, BLK)
    out = pl.pallas_call(
        _body,
        grid=(N // BLK,),
        in_specs=[
            pl.BlockSpec((1, 1, BLK), lambda i: (i, 0, 0)),
            pl.BlockSpec((BLK, D), lambda i: (i, 0)),
            pl.BlockSpec((1, D), lambda i: (0, 0)),
        ],
        out_specs=pl.BlockSpec((NUM_WIN, W), lambda i: (0, 0)),
        out_shape=jax.ShapeDtypeStruct((NUM_WIN, W), jnp.float32),
    )(ids, power_spectrum, weights)
    return out.reshape(NUM_SEGMENTS, 1)


# speedup vs baseline: 6.4725x; 2.9294x over previous
import functools

import jax
import jax.numpy as jnp
from jax import lax
from jax.experimental import pallas as pl
from jax.experimental.pallas import tpu as pltpu
from jax.experimental.pallas import tpu_sc as plsc

N = 320000
D = 128
NUM_SEGMENTS = 4096

BLK = 2560
N_BLKS = N // BLK


def _matvec_body(p_ref, w_ref, y_ref):
    y_ref[0] = lax.dot_general(
        w_ref[...], p_ref[...], (((1,), (1,)), ((), ())),
        preferred_element_type=jnp.float32)


def _matvec(power_spectrum, weights):
    return pl.pallas_call(
        _matvec_body,
        grid=(N_BLKS,),
        in_specs=[
            pl.BlockSpec((BLK, D), lambda i: (i, 0)),
            pl.BlockSpec((1, D), lambda i: (0, 0)),
        ],
        out_specs=pl.BlockSpec((1, 1, BLK), lambda i: (i, 0, 0)),
        out_shape=jax.ShapeDtypeStruct((N_BLKS, 1, BLK), jnp.float32),
    )(power_spectrum, weights)


NC = 2
NS = 16
NW = NC * NS
ROWS_W = N // NW
L = 16
ACC = L * NUM_SEGMENTS


def _seg_body(ids_hbm, y_hbm, out_hbm, ids_v, y_v, acc_v, sem):
    del sem
    c = lax.axis_index("c")
    s = lax.axis_index("s")
    wid = s * NC + c
    base = wid * ROWS_W
    pltpu.sync_copy(ids_hbm.at[pl.ds(base, ROWS_W)], ids_v)
    pltpu.sync_copy(y_hbm.at[pl.ds(base, ROWS_W)], y_v)

    zeros = jnp.zeros((L,), jnp.float32)

    def zero_body(i, _):
        for j in range(16):
            acc_v[pl.ds((i * 16 + j) * L, L)] = zeros
        return 0

    lax.fori_loop(0, ACC // (16 * L), zero_body, 0)

    lane_off = lax.iota(jnp.int32, L) * NUM_SEGMENTS

    def body(i, _):
        ids16 = ids_v[pl.ds(i * L, L)]
        y16 = y_v[pl.ds(i * L, L)]
        plsc.addupdate_scatter(acc_v, [ids16 + lane_off], y16)
        return 0

    lax.fori_loop(0, ROWS_W // L, body, 0)

    pltpu.sync_copy(acc_v, out_hbm.at[wid])


def _segment_scatter(ids, y):
    f = functools.partial(
        pl.kernel,
        out_type=jax.ShapeDtypeStruct((NW, ACC), jnp.float32),
        mesh=plsc.VectorSubcoreMesh(core_axis_name="c", subcore_axis_name="s"),
        scratch_types=[
            pltpu.VMEM((ROWS_W,), jnp.int32),
            pltpu.VMEM((ROWS_W,), jnp.float32),
            pltpu.VMEM((ACC,), jnp.float32),
            pltpu.SemaphoreType.DMA,
        ],
        compiler_params=pltpu.CompilerParams(needs_layout_passes=False),
    )(_seg_body)
    return f(ids, y)


RED_W = 512


def _reduce_body(p_ref, o_ref):
    o_ref[...] = jnp.sum(p_ref[...], axis=0, keepdims=True)


def _reduce(partials):
    return pl.pallas_call(
        _reduce_body,
        grid=(NUM_SEGMENTS // RED_W,),
        in_specs=[pl.BlockSpec((NW * L, RED_W), lambda i: (0, i))],
        out_specs=pl.BlockSpec((1, RED_W), lambda i: (0, i)),
        out_shape=jax.ShapeDtypeStruct((1, NUM_SEGMENTS), jnp.float32),
    )(partials)


def kernel(power_spectrum, all_species, segment_ids, weights):
    del all_species
    ids = segment_ids.astype(jnp.int32)
    y = _matvec(power_spectrum, weights).reshape(N)
    partials = _segment_scatter(ids, y).reshape(NW * L, NUM_SEGMENTS)
    out = _reduce(partials)
    return out.reshape(NUM_SEGMENTS, 1)


# speedup vs baseline: 9.8985x; 1.5293x over previous
import functools

import jax
import jax.numpy as jnp
from jax import lax
from jax.experimental import pallas as pl
from jax.experimental.pallas import tpu as pltpu
from jax.experimental.pallas import tpu_sc as plsc

N = 320000
D = 128
NUM_SEGMENTS = 4096

BLK = 16000
N_BLKS = N // BLK


def _matvec_body(p_ref, w_ref, y_ref):
    y_ref[0] = lax.dot_general(
        w_ref[...], p_ref[...], (((1,), (1,)), ((), ())),
        preferred_element_type=jnp.float32)


def _matvec(power_spectrum, weights):
    return pl.pallas_call(
        _matvec_body,
        grid=(N_BLKS,),
        in_specs=[
            pl.BlockSpec((BLK, D), lambda i: (i, 0)),
            pl.BlockSpec((1, D), lambda i: (0, 0)),
        ],
        out_specs=pl.BlockSpec((1, 1, BLK), lambda i: (i, 0, 0)),
        out_shape=jax.ShapeDtypeStruct((N_BLKS, 1, BLK), jnp.float32),
    )(power_spectrum, weights)


NC = 2
NS = 16
NW = NC * NS
ROWS_W = N // NW
L = 16
ACC = L * NUM_SEGMENTS


def _seg_body(ids_hbm, y_hbm, out_hbm, ids_v, y_v, acc_v, sem):
    del sem
    c = lax.axis_index("c")
    s = lax.axis_index("s")
    wid = s * NC + c
    base = wid * ROWS_W
    pltpu.sync_copy(ids_hbm.at[pl.ds(base, ROWS_W)], ids_v)
    pltpu.sync_copy(y_hbm.at[pl.ds(base, ROWS_W)], y_v)

    zeros = jnp.zeros((L,), jnp.float32)

    def zero_body(i, _):
        for j in range(16):
            acc_v[pl.ds((i * 16 + j) * L, L)] = zeros
        return 0

    lax.fori_loop(0, ACC // (16 * L), zero_body, 0)

    lane_off = lax.iota(jnp.int32, L) * NUM_SEGMENTS

    def body(i, _):
        ids16 = ids_v[pl.ds(i * L, L)]
        y16 = y_v[pl.ds(i * L, L)]
        plsc.addupdate_scatter(acc_v, [ids16 + lane_off], y16)
        return 0

    lax.fori_loop(0, ROWS_W // L, body, 0)

    pltpu.sync_copy(acc_v, out_hbm.at[wid])


def _segment_scatter(ids, y):
    f = functools.partial(
        pl.kernel,
        out_type=jax.ShapeDtypeStruct((NW, ACC), jnp.float32),
        mesh=plsc.VectorSubcoreMesh(core_axis_name="c", subcore_axis_name="s"),
        scratch_types=[
            pltpu.VMEM((ROWS_W,), jnp.int32),
            pltpu.VMEM((ROWS_W,), jnp.float32),
            pltpu.VMEM((ACC,), jnp.float32),
            pltpu.SemaphoreType.DMA,
        ],
        compiler_params=pltpu.CompilerParams(needs_layout_passes=False),
    )(_seg_body)
    return f(ids, y)


RED_W = 512


def _reduce_body(p_ref, o_ref):
    o_ref[...] = jnp.sum(p_ref[...], axis=0, keepdims=True)


def _reduce(partials):
    return pl.pallas_call(
        _reduce_body,
        grid=(NUM_SEGMENTS // RED_W,),
        in_specs=[pl.BlockSpec((NW * L, RED_W), lambda i: (0, i))],
        out_specs=pl.BlockSpec((1, RED_W), lambda i: (0, i)),
        out_shape=jax.ShapeDtypeStruct((1, NUM_SEGMENTS), jnp.float32),
    )(partials)


def kernel(power_spectrum, all_species, segment_ids, weights):
    del all_species
    ids = segment_ids.astype(jnp.int32)
    y = _matvec(power_spectrum, weights).reshape(N)
    partials = _segment_scatter(ids, y).reshape(NW * L, NUM_SEGMENTS)
    out = _reduce(partials)
    return out.reshape(NUM_SEGMENTS, 1)


# speedup vs baseline: 12.0531x; 1.2177x over previous
import functools

import jax
import jax.numpy as jnp
from jax import lax
from jax.experimental import pallas as pl
from jax.experimental.pallas import tpu as pltpu
from jax.experimental.pallas import tpu_sc as plsc

N = 320000
D = 128
NUM_SEGMENTS = 4096

BLK = 16000
N_BLKS = N // BLK


def _matvec_body(p_ref, w_ref, y_ref):
    y_ref[0] = lax.dot_general(
        w_ref[...], p_ref[...], (((1,), (1,)), ((), ())),
        preferred_element_type=jnp.float32)


def _matvec(power_spectrum, weights):
    return pl.pallas_call(
        _matvec_body,
        grid=(N_BLKS,),
        in_specs=[
            pl.BlockSpec((BLK, D), lambda i: (i, 0)),
            pl.BlockSpec((1, D), lambda i: (0, 0)),
        ],
        out_specs=pl.BlockSpec((1, 1, BLK), lambda i: (i, 0, 0)),
        out_shape=jax.ShapeDtypeStruct((N_BLKS, 1, BLK), jnp.float32),
    )(power_spectrum, weights)


NC = 2
NS = 16
NW = NC * NS
ROWS_W = N // NW
L = 16
ACC = L * NUM_SEGMENTS


def _seg_body(ids_hbm, y_hbm, out_hbm, ids_v, y_v, acc_v, res_v, sem1, sem2):
    c = lax.axis_index("c")
    s = lax.axis_index("s")
    wid = s * NC + c
    base = wid * ROWS_W
    cp_ids = pltpu.async_copy(ids_hbm.at[pl.ds(base, ROWS_W)], ids_v, sem1)
    cp_y = pltpu.async_copy(y_hbm.at[pl.ds(base, ROWS_W)], y_v, sem2)

    zeros = jnp.zeros((L,), jnp.float32)

    @plsc.parallel_loop(0, ACC // (16 * L), unroll=2)
    def _(i):
        for j in range(16):
            acc_v[pl.ds((i * 16 + j) * L, L)] = zeros

    cp_ids.wait()
    cp_y.wait()

    lane_off = lax.iota(jnp.int32, L) * NUM_SEGMENTS

    @plsc.parallel_loop(0, ROWS_W // L, unroll=8)
    def _(i):
        ids16 = ids_v[pl.ds(i * L, L)]
        y16 = y_v[pl.ds(i * L, L)]
        plsc.addupdate_scatter(acc_v, [ids16 + lane_off], y16)

    @plsc.parallel_loop(0, NUM_SEGMENTS // L, unroll=4)
    def _(k):
        kb = k * L
        acc = acc_v[pl.ds(kb, L)]
        for r in range(1, L):
            acc = acc + acc_v[pl.ds(r * NUM_SEGMENTS + kb, L)]
        res_v[pl.ds(kb, L)] = acc

    pltpu.sync_copy(res_v, out_hbm.at[wid])


def _segment_scatter(ids, y):
    f = functools.partial(
        pl.kernel,
        out_type=jax.ShapeDtypeStruct((NW, NUM_SEGMENTS), jnp.float32),
        mesh=plsc.VectorSubcoreMesh(core_axis_name="c", subcore_axis_name="s"),
        scratch_types=[
            pltpu.VMEM((ROWS_W,), jnp.int32),
            pltpu.VMEM((ROWS_W,), jnp.float32),
            pltpu.VMEM((ACC,), jnp.float32),
            pltpu.VMEM((NUM_SEGMENTS,), jnp.float32),
            pltpu.SemaphoreType.DMA,
            pltpu.SemaphoreType.DMA,
        ],
        compiler_params=pltpu.CompilerParams(needs_layout_passes=False),
    )(_seg_body)
    return f(ids, y)


RED_W = 2048


def _reduce_body(p_ref, o_ref):
    o_ref[...] = jnp.sum(p_ref[...], axis=0, keepdims=True)


def _reduce(partials):
    return pl.pallas_call(
        _reduce_body,
        grid=(NUM_SEGMENTS // RED_W,),
        in_specs=[pl.BlockSpec((NW, RED_W), lambda i: (0, i))],
        out_specs=pl.BlockSpec((1, RED_W), lambda i: (0, i)),
        out_shape=jax.ShapeDtypeStruct((1, NUM_SEGMENTS), jnp.float32),
    )(partials)


def kernel(power_spectrum, all_species, segment_ids, weights):
    del all_species
    ids = segment_ids.astype(jnp.int32)
    y = _matvec(power_spectrum, weights).reshape(N)
    partials = _segment_scatter(ids, y)
    out = _reduce(partials)
    return out.reshape(NUM_SEGMENTS, 1)
